# Initial kernel scaffold; baseline (speedup 1.0000x reference)
import jax
import jax.numpy as jnp
from jax.experimental import pallas as pl


def kernel(query, memory_keys, memory_values, table, W):
    raise NotImplementedError("write your pallas kernel here")


# speedup vs baseline: 1.3445x; 1.3445x over previous
import functools

import jax
import jax.numpy as jnp
from jax import lax
from jax.experimental import pallas as pl
from jax.experimental.pallas import tpu as pltpu
from jax.experimental.pallas import tpu_sc as plsc

DIM = 64
L = 50
LANES = 16

NUM_WORKERS = 32
ROWS_PER_WORKER = 260
CHUNK_ROWS = 2
TOK_PER_CHUNK = CHUNK_ROWS * L
CHUNKS = ROWS_PER_WORKER // CHUNK_ROWS
N_PAD = NUM_WORKERS * ROWS_PER_WORKER


def _sc_encode(tokens3, table):
  mesh = plsc.VectorSubcoreMesh(core_axis_name="c", subcore_axis_name="s")

  @functools.partial(
      pl.kernel,
      out_type=jax.ShapeDtypeStruct((NUM_WORKERS, ROWS_PER_WORKER, DIM),
                                    jnp.float32),
      mesh=mesh,
      scratch_types=[
          pltpu.VMEM((CHUNKS, TOK_PER_CHUNK), jnp.int32),
          pltpu.VMEM((TOK_PER_CHUNK, DIM), jnp.float32),
          pltpu.VMEM((TOK_PER_CHUNK, DIM), jnp.float32),
          pltpu.VMEM((ROWS_PER_WORKER, DIM), jnp.float32),
          pltpu.SemaphoreType.DMA,
          pltpu.SemaphoreType.DMA,
      ],
      compiler_params=pltpu.CompilerParams(needs_layout_passes=False,
                                           use_tc_tiling_on_sc=False),
  )
  def body(tokens_hbm, table_hbm, out_hbm, idx_v, buf0, buf1, acc_v, sem0,
           sem1):
    wid = lax.axis_index("s") * 2 + lax.axis_index("c")
    pltpu.sync_copy(tokens_hbm.at[wid], idx_v)

    bufs = (buf0, buf1)
    sems = (sem0, sem1)

    pltpu.make_async_copy(table_hbm.at[idx_v.at[0]], buf0, sem0).start()
    pltpu.make_async_copy(table_hbm.at[idx_v.at[1]], buf1, sem1).start()

    inv_l = jnp.float32(1.0 / L)
    last_lane = jnp.full((LANES, 1), LANES - 1, jnp.int32)
    bcast_dnums = lax.GatherDimensionNumbers(
        offset_dims=(), collapsed_slice_dims=(0,), start_index_map=(0,))

    def _bcast_last(x):
      return lax.gather(x, last_lane, bcast_dnums, slice_sizes=(1,),
                        mode=lax.GatherScatterMode.PROMISE_IN_BOUNDS)

    def compute_chunk(c, buf):
      for r in range(CHUNK_ROWS):
        def tok_body(t, accs, r=r):
          a0, a1, a2, a3 = accs
          base = r * L + t
          e0 = buf[base, pl.ds(0, 16)]
          e1 = buf[base, pl.ds(16, 16)]
          e2 = buf[base, pl.ds(32, 16)]
          e3 = buf[base, pl.ds(48, 16)]
          ssv = e0 * e0 + e1 * e1 + e2 * e2 + e3 * e3
          s = _bcast_last(plsc.cumsum(ssv))
          s = jnp.maximum(s, jnp.float32(1e-12))
          i = lax.bitcast_convert_type(s, jnp.int32)
          i = jnp.full((LANES,), 0x5F3759DF,
                       jnp.int32) - lax.shift_right_logical(i, 1)
          y = lax.bitcast_convert_type(i, jnp.float32)
          h = jnp.float32(0.5) * s
          y = y * (jnp.float32(1.5) - h * y * y)
          y = y * (jnp.float32(1.5) - h * y * y)
          y = y * (jnp.float32(1.5) - h * y * y)
          scale = jnp.minimum(jnp.float32(1.0), jnp.float32(10.0) * y)
          return (a0 + e0 * scale, a1 + e1 * scale, a2 + e2 * scale,
                  a3 + e3 * scale)

        z = jnp.zeros((LANES,), jnp.float32)
        a0, a1, a2, a3 = lax.fori_loop(0, L, tok_body, (z, z, z, z))
        row = c * CHUNK_ROWS + r
        acc_v[row, pl.ds(0, 16)] = a0 * inv_l
        acc_v[row, pl.ds(16, 16)] = a1 * inv_l
        acc_v[row, pl.ds(32, 16)] = a2 * inv_l
        acc_v[row, pl.ds(48, 16)] = a3 * inv_l

    def outer(io, carry):
      cc = io * 2
      for b in range(2):
        c = cc + b
        buf, sem = bufs[b], sems[b]
        pltpu.make_async_copy(table_hbm.at[idx_v.at[c]], buf, sem).wait()
        compute_chunk(c, buf)

        @pl.when(c + 2 < CHUNKS)
        def _(c=c, buf=buf, sem=sem):
          pltpu.make_async_copy(table_hbm.at[idx_v.at[c + 2]], buf,
                                sem).start()
      return carry

    lax.fori_loop(0, CHUNKS // 2, outer, 0)

    pltpu.sync_copy(acc_v, out_hbm.at[wid])

  return body(tokens3, table)


def _tc_attend(q, k, v, w):
  m_rows = k.shape[0]

  def body(q_ref, k_ref, v_ref, w_ref, out_ref):
    qv = q_ref[...]
    kv = k_ref[...]
    vv = v_ref[...]
    wv = w_ref[...]
    qn = jnp.maximum(jnp.sqrt(jnp.sum(qv * qv)), 1e-8)
    kn = jnp.maximum(jnp.sqrt(jnp.sum(kv * kv, axis=1, keepdims=True)), 1e-8)
    dots = jnp.sum(qv * kv, axis=1, keepdims=True)
    sim = dots / (qn * kn)
    e = jnp.exp(sim - jnp.max(sim))
    att = e / jnp.sum(e)
    vr = jnp.sum(att * vv, axis=0, keepdims=True)
    res = lax.dot_general(vr, wv, (((1,), (1,)), ((), ())),
                          preferred_element_type=jnp.float32)
    out_ref[...] = jnp.broadcast_to(res, (m_rows, DIM))

  return pl.pallas_call(
      body,
      out_shape=jax.ShapeDtypeStruct((m_rows, DIM), jnp.float32),
  )(q, k, v, w)


def kernel(query, memory_keys, memory_values, table, W):
  m_rows = memory_keys.shape[0]
  tokens = jnp.concatenate(
      [
          memory_keys.astype(jnp.int32),
          memory_values.astype(jnp.int32),
          query.astype(jnp.int32),
      ],
      axis=0,
  )
  tokens = jnp.pad(tokens, ((0, N_PAD - tokens.shape[0]), (0, 0)))
  tokens3 = tokens.reshape(NUM_WORKERS, CHUNKS, TOK_PER_CHUNK)
  enc = _sc_encode(tokens3, table).reshape(N_PAD, DIM)
  k = enc[:m_rows]
  v = enc[m_rows:2 * m_rows]
  q = enc[2 * m_rows:2 * m_rows + 1]
  x_encoded = _tc_attend(q, k, v, W)
  return (x_encoded, v)


# speedup vs baseline: 1.3957x; 1.0381x over previous
import functools

import jax
import jax.numpy as jnp
from jax import lax
from jax.experimental import pallas as pl
from jax.experimental.pallas import tpu as pltpu
from jax.experimental.pallas import tpu_sc as plsc

DIM = 64
L = 50
LANES = 16

NUM_WORKERS = 32
ROWS_PER_WORKER = 264
CHUNK_ROWS = 2
TOK_PER_CHUNK = CHUNK_ROWS * L
CHUNKS = ROWS_PER_WORKER // CHUNK_ROWS
N_PAD = NUM_WORKERS * ROWS_PER_WORKER


def _sc_encode(idx3, table_lin):
  mesh = plsc.VectorSubcoreMesh(core_axis_name="c", subcore_axis_name="s")

  @functools.partial(
      pl.kernel,
      out_type=jax.ShapeDtypeStruct((NUM_WORKERS, ROWS_PER_WORKER, DIM),
                                    jnp.float32),
      mesh=mesh,
      scratch_types=[
          pltpu.VMEM((CHUNKS, TOK_PER_CHUNK), jnp.int32),
          pltpu.VMEM((TOK_PER_CHUNK, DIM), jnp.float32),
          pltpu.VMEM((TOK_PER_CHUNK, DIM), jnp.float32),
          pltpu.VMEM((TOK_PER_CHUNK, DIM), jnp.float32),
          pltpu.VMEM((TOK_PER_CHUNK, DIM), jnp.float32),
          pltpu.VMEM((ROWS_PER_WORKER, DIM), jnp.float32),
          pltpu.SemaphoreType.DMA,
          pltpu.SemaphoreType.DMA,
          pltpu.SemaphoreType.DMA,
          pltpu.SemaphoreType.DMA,
      ],
      compiler_params=pltpu.CompilerParams(needs_layout_passes=False,
                                           use_tc_tiling_on_sc=False),
  )
  def body(idx_hbm, table_hbm, out_hbm, idx_v, buf0, buf1,
           buf2, buf3, acc_v, sem0, sem1, sem2, sem3):
    wid = lax.axis_index("s") * 2 + lax.axis_index("c")
    pltpu.sync_copy(idx_hbm.at[wid], idx_v)

    bufs = (buf0, buf1, buf2, buf3)
    sems = (sem0, sem1, sem2, sem3)
    nbuf = 4

    for b in range(nbuf):
      pltpu.make_async_copy(table_hbm.at[idx_v.at[b]], bufs[b],
                            sems[b]).start()

    inv_l = jnp.float32(1.0 / L)
    last_lane = jnp.full((LANES, 1), LANES - 1, jnp.int32)
    bcast_dnums = lax.GatherDimensionNumbers(
        offset_dims=(), collapsed_slice_dims=(0,), start_index_map=(0,))

    def _bcast_last(x):
      return lax.gather(x, last_lane, bcast_dnums, slice_sizes=(1,),
                        mode=lax.GatherScatterMode.PROMISE_IN_BOUNDS)

    def scaled(buf, base):
      e0 = buf[base, pl.ds(0, 16)]
      e1 = buf[base, pl.ds(16, 16)]
      e2 = buf[base, pl.ds(32, 16)]
      e3 = buf[base, pl.ds(48, 16)]
      ssv = e0 * e0 + e1 * e1 + e2 * e2 + e3 * e3
      s = _bcast_last(plsc.cumsum(ssv))
      s = jnp.maximum(s, jnp.float32(1e-12))
      i = lax.bitcast_convert_type(s, jnp.int32)
      i = jnp.full((LANES,), 0x5F3759DF,
                   jnp.int32) - lax.shift_right_logical(i, 1)
      y = lax.bitcast_convert_type(i, jnp.float32)
      h = jnp.float32(0.5) * s
      y = y * (jnp.float32(1.5) - h * y * y)
      y = y * (jnp.float32(1.5) - h * y * y)
      scale = jnp.minimum(jnp.float32(1.0), jnp.float32(10.0) * y)
      return (e0 * scale, e1 * scale, e2 * scale, e3 * scale)

    def compute_chunk(c, buf):
      for r in range(CHUNK_ROWS):
        def tok_body(t, accs, r=r):
          a = accs
          base = r * L + t * 2
          fa = scaled(buf, base)
          fb = scaled(buf, base + 1)
          return (a[0] + fa[0], a[1] + fa[1], a[2] + fa[2], a[3] + fa[3],
                  a[4] + fb[0], a[5] + fb[1], a[6] + fb[2], a[7] + fb[3])

        z = jnp.zeros((LANES,), jnp.float32)
        acc = lax.fori_loop(0, L // 2, tok_body, (z,) * 8)
        row = c * CHUNK_ROWS + r
        acc_v[row, pl.ds(0, 16)] = (acc[0] + acc[4]) * inv_l
        acc_v[row, pl.ds(16, 16)] = (acc[1] + acc[5]) * inv_l
        acc_v[row, pl.ds(32, 16)] = (acc[2] + acc[6]) * inv_l
        acc_v[row, pl.ds(48, 16)] = (acc[3] + acc[7]) * inv_l

    def outer(io, carry):
      cc = io * nbuf
      for b in range(nbuf):
        c = cc + b
        buf, sem = bufs[b], sems[b]
        pltpu.make_async_copy(table_hbm.at[idx_v.at[c]], buf, sem).wait()
        compute_chunk(c, buf)

        @pl.when(c + nbuf < CHUNKS)
        def _(c=c, buf=buf, sem=sem):
          pltpu.make_async_copy(table_hbm.at[idx_v.at[c + nbuf]], buf,
                                sem).start()
      return carry

    lax.fori_loop(0, CHUNKS // nbuf, outer, 0)

    pltpu.sync_copy(acc_v, out_hbm.at[wid])

  return body(idx3, table_lin)


def _tc_repack(table_t):
  v_rows = table_t.shape[1]
  blk = 512
  grid = (v_rows // 2 + blk - 1) // blk
  hs = grid * blk

  def body(a_ref, b_ref, out_ref):
    out_ref[:, 0:DIM] = a_ref[...].T
    out_ref[:, DIM:2 * DIM] = b_ref[...].T

  return pl.pallas_call(
      body,
      grid=(grid,),
      in_specs=[
          pl.BlockSpec((DIM, blk), lambda i: (0, i)),
          pl.BlockSpec((DIM, blk), lambda i, g=grid: (0, i + g)),
      ],
      out_specs=pl.BlockSpec((blk, 2 * DIM), lambda i: (i, 0)),
      out_shape=jax.ShapeDtypeStruct((hs, 2 * DIM), jnp.float32),
  )(table_t, table_t)


def _tc_attend(q, k, v, w):
  m_rows = k.shape[0]

  def body(q_ref, k_ref, v_ref, w_ref, out_ref):
    qv = q_ref[...]
    kv = k_ref[...]
    vv = v_ref[...]
    wv = w_ref[...]
    qn = jnp.maximum(jnp.sqrt(jnp.sum(qv * qv)), 1e-8)
    kn = jnp.maximum(jnp.sqrt(jnp.sum(kv * kv, axis=1, keepdims=True)), 1e-8)
    dots = jnp.sum(qv * kv, axis=1, keepdims=True)
    sim = dots / (qn * kn)
    e = jnp.exp(sim - jnp.max(sim))
    att = e / jnp.sum(e)
    vr = jnp.sum(att * vv, axis=0, keepdims=True)
    res = lax.dot_general(vr, wv, (((1,), (1,)), ((), ())),
                          preferred_element_type=jnp.float32)
    out_ref[...] = jnp.broadcast_to(res, (m_rows, DIM))

  return pl.pallas_call(
      body,
      out_shape=jax.ShapeDtypeStruct((m_rows, DIM), jnp.float32),
  )(q, k, v, w)


def kernel(query, memory_keys, memory_values, table, W):
  m_rows = memory_keys.shape[0]
  half = ((table.shape[0] // 2 + 511) // 512) * 512
  tokens = jnp.concatenate(
      [
          memory_keys.astype(jnp.int32),
          memory_values.astype(jnp.int32),
          query.astype(jnp.int32),
      ],
      axis=0,
  )
  pad_n = N_PAD - tokens.shape[0]
  padtok = (jnp.arange(pad_n * L, dtype=jnp.int32).reshape(pad_n, L)
            * 613) % jnp.int32(table.shape[0])
  tokens = jnp.concatenate([tokens, padtok], axis=0)
  idx3 = jnp.where(tokens < half, 2 * tokens,
                   2 * (tokens - half) + 1).reshape(NUM_WORKERS, CHUNKS,
                                                    TOK_PER_CHUNK)
  table2 = _tc_repack(table.T)
  table_lin = table2.reshape(2 * half, DIM)
  enc = _sc_encode(idx3, table_lin).reshape(N_PAD, DIM)
  k = enc[:m_rows]
  v = enc[m_rows:2 * m_rows]
  q = enc[2 * m_rows:2 * m_rows + 1]
  x_encoded = _tc_attend(q, k, v, W)
  return (x_encoded, v)


# speedup vs baseline: 2.9779x; 2.1336x over previous
import functools

import jax
import jax.numpy as jnp
from jax import lax
from jax.experimental import pallas as pl
from jax.experimental.pallas import tpu as pltpu
from jax.experimental.pallas import tpu_sc as plsc

DIM = 64
L = 50
LANES = 16

NUM_WORKERS = 32
ROWS_PER_WORKER = 264
CHUNK_ROWS = 2
TOK_PER_CHUNK = CHUNK_ROWS * L
CHUNKS = ROWS_PER_WORKER // CHUNK_ROWS
N_PAD = NUM_WORKERS * ROWS_PER_WORKER
REPACK_BLK = 4096


def _sc_encode(idx3, table_lin):
  mesh = plsc.VectorSubcoreMesh(core_axis_name="c", subcore_axis_name="s")

  @functools.partial(
      pl.kernel,
      out_type=jax.ShapeDtypeStruct((NUM_WORKERS, ROWS_PER_WORKER, DIM),
                                    jnp.float32),
      mesh=mesh,
      scratch_types=[
          pltpu.VMEM((CHUNKS, TOK_PER_CHUNK), jnp.int32),
          pltpu.VMEM((TOK_PER_CHUNK, DIM), jnp.float32),
          pltpu.VMEM((TOK_PER_CHUNK, DIM), jnp.float32),
          pltpu.VMEM((TOK_PER_CHUNK, DIM), jnp.float32),
          pltpu.VMEM((TOK_PER_CHUNK, DIM), jnp.float32),
          pltpu.VMEM((ROWS_PER_WORKER, DIM), jnp.float32),
          pltpu.SemaphoreType.DMA,
          pltpu.SemaphoreType.DMA,
          pltpu.SemaphoreType.DMA,
          pltpu.SemaphoreType.DMA,
      ],
      compiler_params=pltpu.CompilerParams(needs_layout_passes=False,
                                           use_tc_tiling_on_sc=False),
  )
  def body(idx_hbm, table_hbm, out_hbm, idx_v, buf0, buf1,
           buf2, buf3, acc_v, sem0, sem1, sem2, sem3):
    wid = lax.axis_index("s") * 2 + lax.axis_index("c")
    pltpu.sync_copy(idx_hbm.at[wid], idx_v)

    bufs = (buf0, buf1, buf2, buf3)
    sems = (sem0, sem1, sem2, sem3)
    nbuf = 4

    for b in range(nbuf):
      pltpu.make_async_copy(table_hbm.at[idx_v.at[b]], bufs[b],
                            sems[b]).start()

    inv_l = jnp.float32(1.0 / L)
    last_lane = jnp.full((LANES, 1), LANES - 1, jnp.int32)
    bcast_dnums = lax.GatherDimensionNumbers(
        offset_dims=(), collapsed_slice_dims=(0,), start_index_map=(0,))

    def _bcast_last(x):
      return lax.gather(x, last_lane, bcast_dnums, slice_sizes=(1,),
                        mode=lax.GatherScatterMode.PROMISE_IN_BOUNDS)

    def scaled(buf, base):
      e0 = buf[base, pl.ds(0, 16)]
      e1 = buf[base, pl.ds(16, 16)]
      e2 = buf[base, pl.ds(32, 16)]
      e3 = buf[base, pl.ds(48, 16)]
      ssv = e0 * e0 + e1 * e1 + e2 * e2 + e3 * e3
      s = _bcast_last(plsc.cumsum(ssv))
      s = jnp.maximum(s, jnp.float32(1e-12))
      i = lax.bitcast_convert_type(s, jnp.int32)
      i = jnp.full((LANES,), 0x5F3759DF,
                   jnp.int32) - lax.shift_right_logical(i, 1)
      y = lax.bitcast_convert_type(i, jnp.float32)
      h = jnp.float32(0.5) * s
      y = y * (jnp.float32(1.5) - h * y * y)
      y = y * (jnp.float32(1.5) - h * y * y)
      scale = jnp.minimum(jnp.float32(1.0), jnp.float32(10.0) * y)
      return (e0 * scale, e1 * scale, e2 * scale, e3 * scale)

    def compute_chunk(c, buf):
      for r in range(CHUNK_ROWS):
        def tok_body(t, accs, r=r):
          a = accs
          base = r * L + t * 2
          fa = scaled(buf, base)
          fb = scaled(buf, base + 1)
          return (a[0] + fa[0], a[1] + fa[1], a[2] + fa[2], a[3] + fa[3],
                  a[4] + fb[0], a[5] + fb[1], a[6] + fb[2], a[7] + fb[3])

        z = jnp.zeros((LANES,), jnp.float32)
        acc = lax.fori_loop(0, L // 2, tok_body, (z,) * 8)
        row = c * CHUNK_ROWS + r
        acc_v[row, pl.ds(0, 16)] = (acc[0] + acc[4]) * inv_l
        acc_v[row, pl.ds(16, 16)] = (acc[1] + acc[5]) * inv_l
        acc_v[row, pl.ds(32, 16)] = (acc[2] + acc[6]) * inv_l
        acc_v[row, pl.ds(48, 16)] = (acc[3] + acc[7]) * inv_l

    def outer(io, carry):
      cc = io * nbuf
      for b in range(nbuf):
        c = cc + b
        buf, sem = bufs[b], sems[b]
        pltpu.make_async_copy(table_hbm.at[idx_v.at[c]], buf, sem).wait()
        compute_chunk(c, buf)

        @pl.when(c + nbuf < CHUNKS)
        def _(c=c, buf=buf, sem=sem):
          pltpu.make_async_copy(table_hbm.at[idx_v.at[c + nbuf]], buf,
                                sem).start()
      return carry

    lax.fori_loop(0, CHUNKS // nbuf, outer, 0)

    pltpu.sync_copy(acc_v, out_hbm.at[wid])

  return body(idx3, table_lin)


def _tc_repack(table_t):
  v_rows = table_t.shape[1]
  blk = REPACK_BLK
  grid = (v_rows // 2 + blk - 1) // blk
  hs = grid * blk

  def body(a_ref, b_ref, out_ref):
    out_ref[:, 0:DIM] = a_ref[...].T
    out_ref[:, DIM:2 * DIM] = b_ref[...].T

  return pl.pallas_call(
      body,
      grid=(grid,),
      in_specs=[
          pl.BlockSpec((DIM, blk), lambda i: (0, i)),
          pl.BlockSpec(
              (DIM, blk),
              lambda i, g=grid, m=(v_rows - 1) // blk: (0,
                                                        jnp.minimum(i + g, m))),
      ],
      out_specs=pl.BlockSpec((blk, 2 * DIM), lambda i: (i, 0)),
      out_shape=jax.ShapeDtypeStruct((hs, 2 * DIM), jnp.float32),
  )(table_t, table_t)


def _tc_attend(q, k, v, w):
  m_rows = k.shape[0]

  def body(q_ref, k_ref, v_ref, w_ref, out_ref):
    qv = q_ref[...]
    kv = k_ref[...]
    vv = v_ref[...]
    wv = w_ref[...]
    qn = jnp.maximum(jnp.sqrt(jnp.sum(qv * qv)), 1e-8)
    kn = jnp.maximum(jnp.sqrt(jnp.sum(kv * kv, axis=1, keepdims=True)), 1e-8)
    dots = jnp.sum(qv * kv, axis=1, keepdims=True)
    sim = dots / (qn * kn)
    e = jnp.exp(sim - jnp.max(sim))
    att = e / jnp.sum(e)
    vr = jnp.sum(att * vv, axis=0, keepdims=True)
    res = lax.dot_general(vr, wv, (((1,), (1,)), ((), ())),
                          preferred_element_type=jnp.float32)
    out_ref[...] = jnp.broadcast_to(res, (m_rows, DIM))

  return pl.pallas_call(
      body,
      out_shape=jax.ShapeDtypeStruct((m_rows, DIM), jnp.float32),
  )(q, k, v, w)


def kernel(query, memory_keys, memory_values, table, W):
  m_rows = memory_keys.shape[0]
  half = -(-(table.shape[0] // 2) // REPACK_BLK) * REPACK_BLK
  tokens = jnp.concatenate(
      [
          memory_keys.astype(jnp.int32),
          memory_values.astype(jnp.int32),
          query.astype(jnp.int32),
      ],
      axis=0,
  )
  pad_n = N_PAD - tokens.shape[0]
  padtok = (jnp.arange(pad_n * L, dtype=jnp.int32).reshape(pad_n, L)
            * 613) % jnp.int32(table.shape[0])
  tokens = jnp.concatenate([tokens, padtok], axis=0)
  idx3 = jnp.where(tokens < half, 2 * tokens,
                   2 * (tokens - half) + 1).reshape(NUM_WORKERS, CHUNKS,
                                                    TOK_PER_CHUNK)
  table2 = _tc_repack(table.T)
  table_lin = table2.reshape(2 * half, DIM)
  enc = _sc_encode(idx3, table_lin).reshape(N_PAD, DIM)
  k = enc[:m_rows]
  v = enc[m_rows:2 * m_rows]
  q = enc[2 * m_rows:2 * m_rows + 1]
  x_encoded = _tc_attend(q, k, v, W)
  return (x_encoded, v)


# speedup vs baseline: 3.2687x; 1.0977x over previous
import functools

import jax
import jax.numpy as jnp
from jax import lax
from jax.experimental import pallas as pl
from jax.experimental.pallas import tpu as pltpu
from jax.experimental.pallas import tpu_sc as plsc

DIM = 64
L = 50
LANES = 16

NUM_WORKERS = 32
ROWS_PER_WORKER = 264
CHUNK_ROWS = 2
TOK_PER_CHUNK = CHUNK_ROWS * L
CHUNKS = ROWS_PER_WORKER // CHUNK_ROWS
N_PAD = NUM_WORKERS * ROWS_PER_WORKER
REPACK_BLK = 8192


def _sc_encode(idx3, table_lin):
  mesh = plsc.VectorSubcoreMesh(core_axis_name="c", subcore_axis_name="s")

  @functools.partial(
      pl.kernel,
      out_type=jax.ShapeDtypeStruct((NUM_WORKERS, ROWS_PER_WORKER, DIM),
                                    jnp.float32),
      mesh=mesh,
      scratch_types=[
          pltpu.VMEM((CHUNKS, TOK_PER_CHUNK), jnp.int32),
          pltpu.VMEM((TOK_PER_CHUNK, DIM), jnp.float32),
          pltpu.VMEM((TOK_PER_CHUNK, DIM), jnp.float32),
          pltpu.VMEM((TOK_PER_CHUNK, DIM), jnp.float32),
          pltpu.VMEM((TOK_PER_CHUNK, DIM), jnp.float32),
          pltpu.VMEM((ROWS_PER_WORKER, DIM), jnp.float32),
          pltpu.SemaphoreType.DMA,
          pltpu.SemaphoreType.DMA,
          pltpu.SemaphoreType.DMA,
          pltpu.SemaphoreType.DMA,
      ],
      compiler_params=pltpu.CompilerParams(needs_layout_passes=False,
                                           use_tc_tiling_on_sc=False),
  )
  def body(idx_hbm, table_hbm, out_hbm, idx_v, buf0, buf1,
           buf2, buf3, acc_v, sem0, sem1, sem2, sem3):
    wid = lax.axis_index("s") * 2 + lax.axis_index("c")
    pltpu.sync_copy(idx_hbm.at[wid], idx_v)

    bufs = (buf0, buf1, buf2, buf3)
    sems = (sem0, sem1, sem2, sem3)
    nbuf = 4

    for b in range(nbuf):
      pltpu.make_async_copy(table_hbm.at[idx_v.at[b]], bufs[b],
                            sems[b]).start()

    inv_l = jnp.float32(1.0 / L)
    last_lane = jnp.full((LANES, 1), LANES - 1, jnp.int32)
    bcast_dnums = lax.GatherDimensionNumbers(
        offset_dims=(), collapsed_slice_dims=(0,), start_index_map=(0,))

    def _bcast_last(x):
      return lax.gather(x, last_lane, bcast_dnums, slice_sizes=(1,),
                        mode=lax.GatherScatterMode.PROMISE_IN_BOUNDS)

    def scaled(buf, base):
      e0 = buf[base, pl.ds(0, 16)]
      e1 = buf[base, pl.ds(16, 16)]
      e2 = buf[base, pl.ds(32, 16)]
      e3 = buf[base, pl.ds(48, 16)]
      ssv = e0 * e0 + e1 * e1 + e2 * e2 + e3 * e3
      s = _bcast_last(plsc.cumsum(ssv))
      s = jnp.maximum(s, jnp.float32(1e-12))
      i = lax.bitcast_convert_type(s, jnp.int32)
      i = jnp.full((LANES,), 0x5F3759DF,
                   jnp.int32) - lax.shift_right_logical(i, 1)
      y = lax.bitcast_convert_type(i, jnp.float32)
      h = jnp.float32(0.5) * s
      y = y * (jnp.float32(1.5) - h * y * y)
      scale = jnp.minimum(jnp.float32(1.0), jnp.float32(10.0) * y)
      return (e0 * scale, e1 * scale, e2 * scale, e3 * scale)

    def compute_chunk(c, buf):
      for r in range(CHUNK_ROWS):
        def tok_body(t, accs, r=r):
          a = accs
          base = r * L + t * 2
          fa = scaled(buf, base)
          fb = scaled(buf, base + 1)
          return (a[0] + fa[0], a[1] + fa[1], a[2] + fa[2], a[3] + fa[3],
                  a[4] + fb[0], a[5] + fb[1], a[6] + fb[2], a[7] + fb[3])

        z = jnp.zeros((LANES,), jnp.float32)
        acc = lax.fori_loop(0, L // 2, tok_body, (z,) * 8)
        row = c * CHUNK_ROWS + r
        acc_v[row, pl.ds(0, 16)] = (acc[0] + acc[4]) * inv_l
        acc_v[row, pl.ds(16, 16)] = (acc[1] + acc[5]) * inv_l
        acc_v[row, pl.ds(32, 16)] = (acc[2] + acc[6]) * inv_l
        acc_v[row, pl.ds(48, 16)] = (acc[3] + acc[7]) * inv_l

    def outer(io, carry):
      cc = io * nbuf
      for b in range(nbuf):
        c = cc + b
        buf, sem = bufs[b], sems[b]
        pltpu.make_async_copy(table_hbm.at[idx_v.at[c]], buf, sem).wait()
        compute_chunk(c, buf)

        @pl.when(c + nbuf < CHUNKS)
        def _(c=c, buf=buf, sem=sem):
          pltpu.make_async_copy(table_hbm.at[idx_v.at[c + nbuf]], buf,
                                sem).start()
      return carry

    lax.fori_loop(0, CHUNKS // nbuf, outer, 0)

    pltpu.sync_copy(acc_v, out_hbm.at[wid])

  return body(idx3, table_lin)


def _tc_repack(table_t):
  v_rows = table_t.shape[1]
  blk = REPACK_BLK
  grid = (v_rows // 2 + blk - 1) // blk
  hs = grid * blk

  def body(a_ref, b_ref, out_ref):
    out_ref[:, 0:DIM] = a_ref[...].T
    out_ref[:, DIM:2 * DIM] = b_ref[...].T

  return pl.pallas_call(
      body,
      grid=(grid,),
      in_specs=[
          pl.BlockSpec((DIM, blk), lambda i: (0, i)),
          pl.BlockSpec(
              (DIM, blk),
              lambda i, g=grid, m=(v_rows - 1) // blk: (0,
                                                        jnp.minimum(i + g, m))),
      ],
      out_specs=pl.BlockSpec((blk, 2 * DIM), lambda i: (i, 0)),
      out_shape=jax.ShapeDtypeStruct((hs, 2 * DIM), jnp.float32),
  )(table_t, table_t)


def _tc_attend(q, k, v, w):
  m_rows = k.shape[0]

  def body(q_ref, k_ref, v_ref, w_ref, out_ref):
    qv = q_ref[...]
    kv = k_ref[...]
    vv = v_ref[...]
    wv = w_ref[...]
    qn = jnp.maximum(jnp.sqrt(jnp.sum(qv * qv)), 1e-8)
    kn = jnp.maximum(jnp.sqrt(jnp.sum(kv * kv, axis=1, keepdims=True)), 1e-8)
    dots = jnp.sum(qv * kv, axis=1, keepdims=True)
    sim = dots / (qn * kn)
    e = jnp.exp(sim - jnp.max(sim))
    att = e / jnp.sum(e)
    vr = jnp.sum(att * vv, axis=0, keepdims=True)
    res = lax.dot_general(vr, wv, (((1,), (1,)), ((), ())),
                          preferred_element_type=jnp.float32)
    out_ref[...] = jnp.broadcast_to(res, (m_rows, DIM))

  return pl.pallas_call(
      body,
      out_shape=jax.ShapeDtypeStruct((m_rows, DIM), jnp.float32),
  )(q, k, v, w)


def kernel(query, memory_keys, memory_values, table, W):
  m_rows = memory_keys.shape[0]
  half = -(-(table.shape[0] // 2) // REPACK_BLK) * REPACK_BLK
  tokens = jnp.concatenate(
      [
          memory_keys.astype(jnp.int32),
          memory_values.astype(jnp.int32),
          query.astype(jnp.int32),
      ],
      axis=0,
  )
  pad_n = N_PAD - tokens.shape[0]
  padtok = (jnp.arange(pad_n * L, dtype=jnp.int32).reshape(pad_n, L)
            * 613) % jnp.int32(table.shape[0])
  tokens = jnp.concatenate([tokens, padtok], axis=0)
  idx3 = jnp.where(tokens < half, 2 * tokens,
                   2 * (tokens - half) + 1).reshape(NUM_WORKERS, CHUNKS,
                                                    TOK_PER_CHUNK)
  table2 = _tc_repack(table.T)
  table_lin = table2.reshape(2 * half, DIM)
  enc = _sc_encode(idx3, table_lin).reshape(N_PAD, DIM)
  k = enc[:m_rows]
  v = enc[m_rows:2 * m_rows]
  q = enc[2 * m_rows:2 * m_rows + 1]
  x_encoded = _tc_attend(q, k, v, W)
  return (x_encoded, v)


# speedup vs baseline: 3.4060x; 1.0420x over previous
import functools

import jax
import jax.numpy as jnp
from jax import lax
from jax.experimental import pallas as pl
from jax.experimental.pallas import tpu as pltpu
from jax.experimental.pallas import tpu_sc as plsc

DIM = 64
L = 50
LANES = 16

NUM_WORKERS = 32
ROWS_PER_WORKER = 264
CHUNK_ROWS = 2
TOK_PER_CHUNK = CHUNK_ROWS * L
CHUNKS = ROWS_PER_WORKER // CHUNK_ROWS
N_PAD = NUM_WORKERS * ROWS_PER_WORKER
REPACK_BLK = 16384


def _sc_encode(idx3, table_lin):
  mesh = plsc.VectorSubcoreMesh(core_axis_name="c", subcore_axis_name="s")

  @functools.partial(
      pl.kernel,
      out_type=jax.ShapeDtypeStruct((NUM_WORKERS, ROWS_PER_WORKER, DIM),
                                    jnp.float32),
      mesh=mesh,
      scratch_types=[
          pltpu.VMEM((CHUNKS, TOK_PER_CHUNK), jnp.int32),
          pltpu.VMEM((TOK_PER_CHUNK, DIM), jnp.float32),
          pltpu.VMEM((TOK_PER_CHUNK, DIM), jnp.float32),
          pltpu.VMEM((TOK_PER_CHUNK, DIM), jnp.float32),
          pltpu.VMEM((TOK_PER_CHUNK, DIM), jnp.float32),
          pltpu.VMEM((ROWS_PER_WORKER, DIM), jnp.float32),
          pltpu.SemaphoreType.DMA,
          pltpu.SemaphoreType.DMA,
          pltpu.SemaphoreType.DMA,
          pltpu.SemaphoreType.DMA,
      ],
      compiler_params=pltpu.CompilerParams(needs_layout_passes=False,
                                           use_tc_tiling_on_sc=False),
  )
  def body(idx_hbm, table_hbm, out_hbm, idx_v, buf0, buf1,
           buf2, buf3, acc_v, sem0, sem1, sem2, sem3):
    wid = lax.axis_index("s") * 2 + lax.axis_index("c")
    pltpu.sync_copy(idx_hbm.at[wid], idx_v)

    bufs = (buf0, buf1, buf2, buf3)
    sems = (sem0, sem1, sem2, sem3)
    nbuf = 4

    for b in range(nbuf):
      pltpu.make_async_copy(table_hbm.at[idx_v.at[b]], bufs[b],
                            sems[b]).start()

    inv_l = jnp.float32(1.0 / L)
    last_lane = jnp.full((LANES, 1), LANES - 1, jnp.int32)
    bcast_dnums = lax.GatherDimensionNumbers(
        offset_dims=(), collapsed_slice_dims=(0,), start_index_map=(0,))

    def _bcast_last(x):
      return lax.gather(x, last_lane, bcast_dnums, slice_sizes=(1,),
                        mode=lax.GatherScatterMode.PROMISE_IN_BOUNDS)

    def scaled(buf, base):
      e0 = buf[base, pl.ds(0, 16)]
      e1 = buf[base, pl.ds(16, 16)]
      e2 = buf[base, pl.ds(32, 16)]
      e3 = buf[base, pl.ds(48, 16)]
      ssv = e0 * e0 + e1 * e1 + e2 * e2 + e3 * e3
      s = _bcast_last(plsc.cumsum(ssv))
      s = jnp.maximum(s, jnp.float32(1e-12))
      i = lax.bitcast_convert_type(s, jnp.int32)
      i = jnp.full((LANES,), 0x5F3759DF,
                   jnp.int32) - lax.shift_right_logical(i, 1)
      y = lax.bitcast_convert_type(i, jnp.float32)
      h = jnp.float32(0.5) * s
      y = y * (jnp.float32(1.5) - h * y * y)
      scale = jnp.minimum(jnp.float32(1.0), jnp.float32(10.0) * y)
      return (e0 * scale, e1 * scale, e2 * scale, e3 * scale)

    def compute_chunk(c, buf):
      for r in range(CHUNK_ROWS):
        def tok_body(t, accs, r=r):
          a = accs
          base = r * L + t * 2
          fa = scaled(buf, base)
          fb = scaled(buf, base + 1)
          return (a[0] + fa[0], a[1] + fa[1], a[2] + fa[2], a[3] + fa[3],
                  a[4] + fb[0], a[5] + fb[1], a[6] + fb[2], a[7] + fb[3])

        z = jnp.zeros((LANES,), jnp.float32)
        acc = lax.fori_loop(0, L // 2, tok_body, (z,) * 8)
        row = c * CHUNK_ROWS + r
        acc_v[row, pl.ds(0, 16)] = (acc[0] + acc[4]) * inv_l
        acc_v[row, pl.ds(16, 16)] = (acc[1] + acc[5]) * inv_l
        acc_v[row, pl.ds(32, 16)] = (acc[2] + acc[6]) * inv_l
        acc_v[row, pl.ds(48, 16)] = (acc[3] + acc[7]) * inv_l

    def outer(io, carry):
      cc = io * nbuf
      for b in range(nbuf):
        c = cc + b
        buf, sem = bufs[b], sems[b]
        pltpu.make_async_copy(table_hbm.at[idx_v.at[c]], buf, sem).wait()
        compute_chunk(c, buf)

        @pl.when(c + nbuf < CHUNKS)
        def _(c=c, buf=buf, sem=sem):
          pltpu.make_async_copy(table_hbm.at[idx_v.at[c + nbuf]], buf,
                                sem).start()
      return carry

    lax.fori_loop(0, CHUNKS // nbuf, outer, 0)

    pltpu.sync_copy(acc_v, out_hbm.at[wid])

  return body(idx3, table_lin)


def _tc_repack(table_t):
  v_rows = table_t.shape[1]
  blk = REPACK_BLK
  grid = (v_rows // 2 + blk - 1) // blk
  hs = grid * blk

  def body(a_ref, b_ref, out_ref):
    out_ref[:, 0:DIM] = a_ref[...].T
    out_ref[:, DIM:2 * DIM] = b_ref[...].T

  return pl.pallas_call(
      body,
      grid=(grid,),
      in_specs=[
          pl.BlockSpec((DIM, blk), lambda i: (0, i)),
          pl.BlockSpec(
              (DIM, blk),
              lambda i, g=grid, m=(v_rows - 1) // blk: (0,
                                                        jnp.minimum(i + g, m))),
      ],
      out_specs=pl.BlockSpec((blk, 2 * DIM), lambda i: (i, 0)),
      out_shape=jax.ShapeDtypeStruct((hs, 2 * DIM), jnp.float32),
  )(table_t, table_t)


def _tc_attend(q, k, v, w):
  m_rows = k.shape[0]

  def body(q_ref, k_ref, v_ref, w_ref, out_ref):
    qv = q_ref[...]
    kv = k_ref[...]
    vv = v_ref[...]
    wv = w_ref[...]
    qn = jnp.maximum(jnp.sqrt(jnp.sum(qv * qv)), 1e-8)
    kn = jnp.maximum(jnp.sqrt(jnp.sum(kv * kv, axis=1, keepdims=True)), 1e-8)
    dots = jnp.sum(qv * kv, axis=1, keepdims=True)
    sim = dots / (qn * kn)
    e = jnp.exp(sim - jnp.max(sim))
    att = e / jnp.sum(e)
    vr = jnp.sum(att * vv, axis=0, keepdims=True)
    res = lax.dot_general(vr, wv, (((1,), (1,)), ((), ())),
                          preferred_element_type=jnp.float32)
    out_ref[...] = jnp.broadcast_to(res, (m_rows, DIM))

  return pl.pallas_call(
      body,
      out_shape=jax.ShapeDtypeStruct((m_rows, DIM), jnp.float32),
  )(q, k, v, w)


def kernel(query, memory_keys, memory_values, table, W):
  m_rows = memory_keys.shape[0]
  half = -(-(table.shape[0] // 2) // REPACK_BLK) * REPACK_BLK
  tokens = jnp.concatenate(
      [
          memory_keys.astype(jnp.int32),
          memory_values.astype(jnp.int32),
          query.astype(jnp.int32),
      ],
      axis=0,
  )
  pad_n = N_PAD - tokens.shape[0]
  padtok = (jnp.arange(pad_n * L, dtype=jnp.int32).reshape(pad_n, L)
            * 613) % jnp.int32(table.shape[0])
  tokens = jnp.concatenate([tokens, padtok], axis=0)
  idx3 = jnp.where(tokens < half, 2 * tokens,
                   2 * (tokens - half) + 1).reshape(NUM_WORKERS, CHUNKS,
                                                    TOK_PER_CHUNK)
  table2 = _tc_repack(table.T)
  table_lin = table2.reshape(2 * half, DIM)
  enc = _sc_encode(idx3, table_lin).reshape(N_PAD, DIM)
  k = enc[:m_rows]
  v = enc[m_rows:2 * m_rows]
  q = enc[2 * m_rows:2 * m_rows + 1]
  x_encoded = _tc_attend(q, k, v, W)
  return (x_encoded, v)
